# baseline (speedup 1.0000x reference)
import functools
import math

import jax
import jax.numpy as jnp
from jax import lax
from jax.experimental import pallas as pl
from jax.experimental.pallas import tpu as pltpu
from jax.experimental.pallas import tpu_sc as plsc

H = 16
W = 128
F32 = jnp.float32
BF16 = jnp.bfloat16


def _proj_body(k_ref, v_ref, wk_ref, bk_ref, wv_ref, bv_ref, kp_ref, vp_ref):
    kp_ref[...] = jax.lax.dot_general(
        k_ref[...], wk_ref[...], (((1,), (1,)), ((), ())),
        preferred_element_type=F32) + bk_ref[...]
    vp_ref[...] = (jax.lax.dot_general(
        v_ref[...], wv_ref[...], (((1,), (1,)), ((), ())),
        preferred_element_type=F32) + bv_ref[...]).astype(BF16)


def _window(b, lo_ref, hi_ref, i, sp):
    lo8 = pl.multiple_of(
        jnp.minimum((lo_ref[i] // 8) * 8, sp - W), 8)
    fits = hi_ref[i] - lo8 < W
    seg_w = lo8 + jax.lax.broadcasted_iota(jnp.int32, (b.shape[0], W), 1)
    oh_w = (b[:, None] == seg_w).astype(BF16)
    return lo8, fits, oh_w


def _sc_gather(table, idx, nw, ch):
    b_total = idx.shape[0]
    dmr = table.shape[1]
    b_per_w = b_total // nw
    steps = b_per_w // ch
    mesh = plsc.VectorSubcoreMesh(core_axis_name="c", subcore_axis_name="s")

    @functools.partial(
        pl.kernel, mesh=mesh,
        out_type=jax.ShapeDtypeStruct((b_total, dmr), table.dtype),
        scratch_types=[
            pltpu.VMEM((ch,), jnp.int32),
            pltpu.VMEM((ch, dmr), table.dtype),
            pltpu.SemaphoreType.DMA,
        ],
    )
    def gk(table_hbm, idx_hbm, out_hbm, idx_v, rows_v, sem):
        wid = lax.axis_index("s") * 2 + lax.axis_index("c")
        base = wid * b_per_w
        for it in range(steps):
            off = base + it * ch
            pltpu.sync_copy(idx_hbm.at[pl.ds(off, ch)], idx_v)
            pltpu.async_copy(table_hbm.at[idx_v], rows_v, sem).wait()
            pltpu.sync_copy(rows_v, out_hbm.at[pl.ds(off, ch)])

    return gk(table, idx)


def _pass1_body(scale, R, n, lo_ref, hi_ref, q_ref, b_ref, kx_ref, wq_ref,
                bq_ref, ex_ref, den_ref):
    i = pl.program_id(0)
    qp = jax.lax.dot_general(
        q_ref[...].astype(BF16), wq_ref[...], (((1,), (1,)), ((), ())),
        preferred_element_type=F32) + bq_ref[...]
    b = b_ref[0, 0, :]
    sp = den_ref.shape[0]
    row = i * R + jax.lax.broadcasted_iota(jnp.int32, (R, 1), 0)
    lo8, fits, oh_w = _window(b, lo_ref, hi_ref, i, sp)
    kx = kx_ref[...].astype(F32)
    ex = jnp.where(row < n, jnp.exp(qp * kx * scale), 0.0).astype(BF16)
    ex_ref[...] = ex

    @pl.when(i == 0)
    def _init():
        den_ref[...] = jnp.zeros_like(den_ref)

    @pl.when(fits)
    def _fast():
        den_ref[pl.ds(lo8, W), :] += jax.lax.dot_general(
            oh_w, ex, (((0,), (0,)), ((), ())), preferred_element_type=F32)

    @pl.when(jnp.logical_not(fits))
    def _slow():
        seg = jax.lax.broadcasted_iota(jnp.int32, (b.shape[0], sp), 1)
        oh = (b[:, None] == seg).astype(BF16)
        den_ref[...] += jax.lax.dot_general(
            oh, ex, (((0,), (0,)), ((), ())), preferred_element_type=F32)


def _pass2_body(lo_ref, hi_ref, ex_ref, b_ref, vp_ref, den_ref, wo_ref,
                bo_ref, out_ref):
    i = pl.program_id(0)
    b = b_ref[0, 0, :]
    sp = vp_ref.shape[0]
    lo8, fits, oh_w = _window(b, lo_ref, hi_ref, i, sp)

    def emit(wx):
        y = (ex_ref[...].astype(F32) * wx).astype(BF16)
        out_ref[...] = jax.lax.dot_general(
            y, wo_ref[...], (((1,), (1,)), ((), ())),
            preferred_element_type=F32) + bo_ref[...]

    @pl.when(fits)
    def _fast():
        den = den_ref[pl.ds(lo8, W), :]
        w = jnp.where(den > 0.0,
                      vp_ref[pl.ds(lo8, W), :].astype(F32) / den,
                      0.0).astype(BF16)
        emit(jnp.dot(oh_w, w, preferred_element_type=F32))

    @pl.when(jnp.logical_not(fits))
    def _slow():
        den = den_ref[...]
        w = jnp.where(den > 0.0, vp_ref[...].astype(F32) / den,
                      0.0).astype(BF16)
        seg = jax.lax.broadcasted_iota(jnp.int32, (b.shape[0], sp), 1)
        oh = (b[:, None] == seg).astype(BF16)
        emit(jnp.dot(oh, w, preferred_element_type=F32))


def kernel(q, k, v, batch, Wq, bq, Wk, bk, Wv, bv, Wo, bo):
    n, dm = q.shape
    s = k.shape[0]
    d = dm // H
    scale = 1.0 / math.sqrt(float(d))

    R = 512
    nb = -(-n // R)
    npad = nb * R
    sp = -(-s // 128) * 128

    bi = batch.astype(jnp.int32)
    bz = jnp.pad(bi, (0, npad - n), mode="edge")
    b3 = bz.reshape(nb, 1, R)
    b2 = bz.reshape(nb, R)
    lo = b2[:, 0]
    hi = b2[:, R - 1]
    kz = jnp.pad(k, ((0, sp - s), (0, 0)))
    vz = jnp.pad(v, ((0, sp - s), (0, 0)))
    bq2, bk2, bv2, bo2 = (x.reshape(1, dm) for x in (bq, bk, bv, bo))

    full = lambda *shape: pl.BlockSpec(shape, lambda i: (0,) * len(shape))
    smem = pl.BlockSpec(memory_space=pltpu.SMEM)

    kp, vp = pl.pallas_call(
        _proj_body,
        grid=(1,),
        in_specs=[full(sp, dm), full(sp, dm), full(dm, dm), full(1, dm),
                  full(dm, dm), full(1, dm)],
        out_specs=[full(sp, dm), full(sp, dm)],
        out_shape=[jax.ShapeDtypeStruct((sp, dm), F32),
                   jax.ShapeDtypeStruct((sp, dm), BF16)],
    )(kz, vz, Wk, bk2, Wv, bv2)

    kpx = _sc_gather(kp, bz, 32, 112)

    ex, den = pl.pallas_call(
        functools.partial(_pass1_body, scale, R, n),
        grid=(nb,),
        in_specs=[
            smem, smem,
            pl.BlockSpec((R, dm), lambda i: (i, 0)),
            pl.BlockSpec((1, 1, R), lambda i: (i, 0, 0)),
            pl.BlockSpec((R, dm), lambda i: (i, 0)),
            full(dm, dm), full(1, dm),
        ],
        out_specs=[pl.BlockSpec((R, dm), lambda i: (i, 0)), full(sp, dm)],
        out_shape=[jax.ShapeDtypeStruct((n, dm), BF16),
                   jax.ShapeDtypeStruct((sp, dm), F32)],
        compiler_params=pltpu.CompilerParams(
            dimension_semantics=("arbitrary",)),
    )(lo, hi, q, b3, kpx, Wq.astype(BF16), bq2)

    out = pl.pallas_call(
        functools.partial(_pass2_body),
        grid=(nb,),
        in_specs=[
            smem, smem,
            pl.BlockSpec((R, dm), lambda i: (i, 0)),
            pl.BlockSpec((1, 1, R), lambda i: (i, 0, 0)),
            full(sp, dm), full(sp, dm), full(dm, dm), full(1, dm),
        ],
        out_specs=pl.BlockSpec((R, dm), lambda i: (i, 0)),
        out_shape=jax.ShapeDtypeStruct((n, dm), F32),
        compiler_params=pltpu.CompilerParams(
            dimension_semantics=("arbitrary",)),
    )(lo, hi, ex, b3, vp, den, Wo.astype(BF16), bo2)

    return out


# speedup vs baseline: 1.9118x; 1.9118x over previous
import functools
import math

import jax
import jax.numpy as jnp
from jax.experimental import pallas as pl
from jax.experimental.pallas import tpu as pltpu

H = 16
W = 64
F32 = jnp.float32
BF16 = jnp.bfloat16


def _proj_body(k_ref, v_ref, wk_ref, bk_ref, wv_ref, bv_ref, kp_ref, vp_ref):
    kp_ref[...] = (jax.lax.dot_general(
        k_ref[...], wk_ref[...], (((1,), (1,)), ((), ())),
        preferred_element_type=F32) + bk_ref[...]).astype(BF16)
    vp_ref[...] = (jax.lax.dot_general(
        v_ref[...], wv_ref[...], (((1,), (1,)), ((), ())),
        preferred_element_type=F32) + bv_ref[...]).astype(BF16)


def _window(b, lo_ref, hi_ref, i, sp):
    lo8 = pl.multiple_of(
        jnp.minimum((lo_ref[i] // 8) * 8, sp - W), 8)
    fits = hi_ref[i] - lo8 < W
    seg_w = lo8 + jax.lax.broadcasted_iota(jnp.int32, (b.shape[0], W), 1)
    oh_w = (b[:, None] == seg_w).astype(BF16)
    return lo8, fits, oh_w


def _pass1_body(scale, R, n, lo_ref, hi_ref, q_ref, b_ref, wq_ref, bq_ref,
                kp_ref, ex_ref, den_ref):
    i = pl.program_id(0)
    qp = jax.lax.dot_general(
        q_ref[...].astype(BF16), wq_ref[...], (((1,), (1,)), ((), ())),
        preferred_element_type=F32) + bq_ref[...]
    b = b_ref[0, 0, :]
    sp = kp_ref.shape[0]
    row = i * R + jax.lax.broadcasted_iota(jnp.int32, (R, 1), 0)
    lo8, fits, oh_w = _window(b, lo_ref, hi_ref, i, sp)

    @pl.when(i == 0)
    def _init():
        den_ref[...] = jnp.zeros_like(den_ref)

    @pl.when(fits)
    def _fast():
        kx = jnp.dot(oh_w, kp_ref[pl.ds(lo8, W), :],
                     preferred_element_type=F32)
        ex = jnp.where(row < n, jnp.exp(qp * kx * scale), 0.0).astype(BF16)
        ex_ref[...] = ex
        den_ref[pl.ds(lo8, W), :] += jax.lax.dot_general(
            oh_w, ex, (((0,), (0,)), ((), ())), preferred_element_type=F32)

    @pl.when(jnp.logical_not(fits))
    def _slow():
        seg = jax.lax.broadcasted_iota(jnp.int32, (b.shape[0], sp), 1)
        oh = (b[:, None] == seg).astype(BF16)
        kx = jnp.dot(oh, kp_ref[...], preferred_element_type=F32)
        ex = jnp.where(row < n, jnp.exp(qp * kx * scale), 0.0).astype(BF16)
        ex_ref[...] = ex
        den_ref[...] += jax.lax.dot_general(
            oh, ex, (((0,), (0,)), ((), ())), preferred_element_type=F32)


def _pass2_body(lo_ref, hi_ref, ex_ref, b_ref, vp_ref, den_ref, wo_ref,
                bo_ref, out_ref):
    i = pl.program_id(0)
    b = b_ref[0, 0, :]
    sp = vp_ref.shape[0]
    lo8, fits, oh_w = _window(b, lo_ref, hi_ref, i, sp)

    def emit(wx):
        y = (ex_ref[...].astype(F32) * wx).astype(BF16)
        out_ref[...] = jax.lax.dot_general(
            y, wo_ref[...], (((1,), (1,)), ((), ())),
            preferred_element_type=F32) + bo_ref[...]

    @pl.when(fits)
    def _fast():
        den = den_ref[pl.ds(lo8, W), :]
        w = jnp.where(den > 0.0,
                      vp_ref[pl.ds(lo8, W), :].astype(F32) / den,
                      0.0).astype(BF16)
        emit(jnp.dot(oh_w, w, preferred_element_type=F32))

    @pl.when(jnp.logical_not(fits))
    def _slow():
        den = den_ref[...]
        w = jnp.where(den > 0.0, vp_ref[...].astype(F32) / den,
                      0.0).astype(BF16)
        seg = jax.lax.broadcasted_iota(jnp.int32, (b.shape[0], sp), 1)
        oh = (b[:, None] == seg).astype(BF16)
        emit(jnp.dot(oh, w, preferred_element_type=F32))


def kernel(q, k, v, batch, Wq, bq, Wk, bk, Wv, bv, Wo, bo):
    n, dm = q.shape
    s = k.shape[0]
    d = dm // H
    scale = 1.0 / math.sqrt(float(d))

    R = 512
    nb = -(-n // R)
    npad = nb * R
    sp = -(-s // 128) * 128

    bi = batch.astype(jnp.int32)
    bz = jnp.pad(bi, (0, npad - n), mode="edge")
    b3 = bz.reshape(nb, 1, R)
    b2 = bz.reshape(nb, R)
    lo = b2[:, 0]
    hi = b2[:, R - 1]
    kz = jnp.pad(k, ((0, sp - s), (0, 0)))
    vz = jnp.pad(v, ((0, sp - s), (0, 0)))
    bq2, bk2, bv2, bo2 = (x.reshape(1, dm) for x in (bq, bk, bv, bo))

    full = lambda *shape: pl.BlockSpec(shape, lambda i: (0,) * len(shape))
    smem = pl.BlockSpec(memory_space=pltpu.SMEM)

    kp, vp = pl.pallas_call(
        _proj_body,
        grid=(1,),
        in_specs=[full(sp, dm), full(sp, dm), full(dm, dm), full(1, dm),
                  full(dm, dm), full(1, dm)],
        out_specs=[full(sp, dm), full(sp, dm)],
        out_shape=[jax.ShapeDtypeStruct((sp, dm), BF16),
                   jax.ShapeDtypeStruct((sp, dm), BF16)],
    )(kz, vz, Wk, bk2, Wv, bv2)

    ex, den = pl.pallas_call(
        functools.partial(_pass1_body, scale, R, n),
        grid=(nb,),
        in_specs=[
            smem, smem,
            pl.BlockSpec((R, dm), lambda i: (i, 0)),
            pl.BlockSpec((1, 1, R), lambda i: (i, 0, 0)),
            full(dm, dm), full(1, dm), full(sp, dm),
        ],
        out_specs=[pl.BlockSpec((R, dm), lambda i: (i, 0)), full(sp, dm)],
        out_shape=[jax.ShapeDtypeStruct((n, dm), BF16),
                   jax.ShapeDtypeStruct((sp, dm), F32)],
        compiler_params=pltpu.CompilerParams(
            dimension_semantics=("arbitrary",)),
    )(lo, hi, q, b3, Wq.astype(BF16), bq2, kp)

    out = pl.pallas_call(
        functools.partial(_pass2_body),
        grid=(nb,),
        in_specs=[
            smem, smem,
            pl.BlockSpec((R, dm), lambda i: (i, 0)),
            pl.BlockSpec((1, 1, R), lambda i: (i, 0, 0)),
            full(sp, dm), full(sp, dm), full(dm, dm), full(1, dm),
        ],
        out_specs=pl.BlockSpec((R, dm), lambda i: (i, 0)),
        out_shape=jax.ShapeDtypeStruct((n, dm), F32),
        compiler_params=pltpu.CompilerParams(
            dimension_semantics=("arbitrary",)),
    )(lo, hi, ex, b3, vp, den, Wo.astype(BF16), bo2)

    return out


# speedup vs baseline: 2.5491x; 1.3333x over previous
import functools
import math

import jax
import jax.numpy as jnp
from jax.experimental import pallas as pl
from jax.experimental.pallas import tpu as pltpu

H = 16
W = 64
F32 = jnp.float32
BF16 = jnp.bfloat16


def _proj_body(k_ref, v_ref, wk_ref, bk_ref, wv_ref, bv_ref, kp_ref, vp_ref):
    kp_ref[...] = (jax.lax.dot_general(
        k_ref[...], wk_ref[...], (((1,), (1,)), ((), ())),
        preferred_element_type=F32) + bk_ref[...]).astype(BF16)
    vp_ref[...] = (jax.lax.dot_general(
        v_ref[...], wv_ref[...], (((1,), (1,)), ((), ())),
        preferred_element_type=F32) + bv_ref[...]).astype(BF16)


def _window(b, lo_ref, hi_ref, i, sp):
    lo8 = pl.multiple_of(
        jnp.minimum((lo_ref[i] // 8) * 8, sp - W), 8)
    fits = hi_ref[i] - lo8 < W
    seg_w = lo8 + jax.lax.broadcasted_iota(jnp.int32, (b.shape[0], W), 1)
    oh_w = (b[:, None] == seg_w).astype(BF16)
    return lo8, fits, oh_w


def _pass1_body(scale, R, n, lo_ref, hi_ref, q_ref, b_ref, wq_ref, bq_ref,
                kp_ref, ex_ref, den_ref):
    i = pl.program_id(0)
    qp = jax.lax.dot_general(
        q_ref[...].astype(BF16), wq_ref[...], (((1,), (1,)), ((), ())),
        preferred_element_type=F32) + bq_ref[...]
    b = b_ref[0, 0, :]
    sp = kp_ref.shape[0]
    row = i * R + jax.lax.broadcasted_iota(jnp.int32, (R, 1), 0)
    lo8, fits, oh_w = _window(b, lo_ref, hi_ref, i, sp)

    @pl.when(i == 0)
    def _init():
        den_ref[...] = jnp.zeros_like(den_ref)

    @pl.when(fits)
    def _fast():
        kx = jnp.dot(oh_w, kp_ref[pl.ds(lo8, W), :],
                     preferred_element_type=F32)
        ex = jnp.where(row < n, jnp.exp(qp * kx * scale), 0.0).astype(BF16)
        ex_ref[...] = ex
        den_ref[pl.ds(lo8, W), :] += jax.lax.dot_general(
            oh_w, ex, (((0,), (0,)), ((), ())), preferred_element_type=F32)

    @pl.when(jnp.logical_not(fits))
    def _slow():
        seg = jax.lax.broadcasted_iota(jnp.int32, (b.shape[0], sp), 1)
        oh = (b[:, None] == seg).astype(BF16)
        kx = jnp.dot(oh, kp_ref[...], preferred_element_type=F32)
        ex = jnp.where(row < n, jnp.exp(qp * kx * scale), 0.0).astype(BF16)
        ex_ref[...] = ex
        den_ref[...] += jax.lax.dot_general(
            oh, ex, (((0,), (0,)), ((), ())), preferred_element_type=F32)


def _pass2_body(lo_ref, hi_ref, ex_ref, b_ref, vp_ref, den_ref, wo_ref,
                bo_ref, out_ref):
    i = pl.program_id(0)
    b = b_ref[0, 0, :]
    sp = vp_ref.shape[0]
    lo8, fits, oh_w = _window(b, lo_ref, hi_ref, i, sp)

    def emit(wx):
        y = (ex_ref[...].astype(F32) * wx).astype(BF16)
        out_ref[...] = jax.lax.dot_general(
            y, wo_ref[...], (((1,), (1,)), ((), ())),
            preferred_element_type=F32) + bo_ref[...]

    @pl.when(fits)
    def _fast():
        den = den_ref[pl.ds(lo8, W), :]
        w = jnp.where(den > 0.0,
                      vp_ref[pl.ds(lo8, W), :].astype(F32) / den,
                      0.0).astype(BF16)
        emit(jnp.dot(oh_w, w, preferred_element_type=F32))

    @pl.when(jnp.logical_not(fits))
    def _slow():
        den = den_ref[...]
        w = jnp.where(den > 0.0, vp_ref[...].astype(F32) / den,
                      0.0).astype(BF16)
        seg = jax.lax.broadcasted_iota(jnp.int32, (b.shape[0], sp), 1)
        oh = (b[:, None] == seg).astype(BF16)
        emit(jnp.dot(oh, w, preferred_element_type=F32))


def kernel(q, k, v, batch, Wq, bq, Wk, bk, Wv, bv, Wo, bo):
    n, dm = q.shape
    s = k.shape[0]
    d = dm // H
    scale = 1.0 / math.sqrt(float(d))

    R = 1024
    nb = -(-n // R)
    npad = nb * R
    sp = -(-s // 128) * 128

    bi = batch.astype(jnp.int32)
    bz = jnp.pad(bi, (0, npad - n), mode="edge")
    b3 = bz.reshape(nb, 1, R)
    b2 = bz.reshape(nb, R)
    lo = b2[:, 0]
    hi = b2[:, R - 1]
    kz = jnp.pad(k, ((0, sp - s), (0, 0)))
    vz = jnp.pad(v, ((0, sp - s), (0, 0)))
    bq2, bk2, bv2, bo2 = (x.reshape(1, dm) for x in (bq, bk, bv, bo))

    full = lambda *shape: pl.BlockSpec(shape, lambda i: (0,) * len(shape))
    smem = pl.BlockSpec(memory_space=pltpu.SMEM)

    kp, vp = pl.pallas_call(
        _proj_body,
        grid=(1,),
        in_specs=[full(sp, dm), full(sp, dm), full(dm, dm), full(1, dm),
                  full(dm, dm), full(1, dm)],
        out_specs=[full(sp, dm), full(sp, dm)],
        out_shape=[jax.ShapeDtypeStruct((sp, dm), BF16),
                   jax.ShapeDtypeStruct((sp, dm), BF16)],
    )(kz, vz, Wk, bk2, Wv, bv2)

    ex, den = pl.pallas_call(
        functools.partial(_pass1_body, scale, R, n),
        grid=(nb,),
        in_specs=[
            smem, smem,
            pl.BlockSpec((R, dm), lambda i: (i, 0)),
            pl.BlockSpec((1, 1, R), lambda i: (i, 0, 0)),
            full(dm, dm), full(1, dm), full(sp, dm),
        ],
        out_specs=[pl.BlockSpec((R, dm), lambda i: (i, 0)), full(sp, dm)],
        out_shape=[jax.ShapeDtypeStruct((n, dm), BF16),
                   jax.ShapeDtypeStruct((sp, dm), F32)],
        compiler_params=pltpu.CompilerParams(
            dimension_semantics=("arbitrary",)),
    )(lo, hi, q, b3, Wq.astype(BF16), bq2, kp)

    out = pl.pallas_call(
        functools.partial(_pass2_body),
        grid=(nb,),
        in_specs=[
            smem, smem,
            pl.BlockSpec((R, dm), lambda i: (i, 0)),
            pl.BlockSpec((1, 1, R), lambda i: (i, 0, 0)),
            full(sp, dm), full(sp, dm), full(dm, dm), full(1, dm),
        ],
        out_specs=pl.BlockSpec((R, dm), lambda i: (i, 0)),
        out_shape=jax.ShapeDtypeStruct((n, dm), F32),
        compiler_params=pltpu.CompilerParams(
            dimension_semantics=("arbitrary",)),
    )(lo, hi, ex, b3, vp, den, Wo.astype(BF16), bo2)

    return out


# speedup vs baseline: 2.9932x; 1.1742x over previous
import functools
import math

import jax
import jax.numpy as jnp
from jax.experimental import pallas as pl
from jax.experimental.pallas import tpu as pltpu

H = 16
W = 128
F32 = jnp.float32
BF16 = jnp.bfloat16


def _proj_body(k_ref, v_ref, wk_ref, bk_ref, wv_ref, bv_ref, kp_ref, vp_ref):
    kp_ref[...] = (jax.lax.dot_general(
        k_ref[...], wk_ref[...], (((1,), (1,)), ((), ())),
        preferred_element_type=F32) + bk_ref[...]).astype(BF16)
    vp_ref[...] = (jax.lax.dot_general(
        v_ref[...], wv_ref[...], (((1,), (1,)), ((), ())),
        preferred_element_type=F32) + bv_ref[...]).astype(BF16)


def _window(b, lo_ref, hi_ref, i, sp):
    lo8 = pl.multiple_of(
        jnp.minimum((lo_ref[i] // 8) * 8, sp - W), 8)
    fits = hi_ref[i] - lo8 < W
    seg_w = lo8 + jax.lax.broadcasted_iota(jnp.int32, (b.shape[0], W), 1)
    oh_w = (b[:, None] == seg_w).astype(BF16)
    return lo8, fits, oh_w


def _pass1_body(scale, R, n, lo_ref, hi_ref, q_ref, b_ref, wq_ref, bq_ref,
                kp_ref, ex_ref, den_ref):
    i = pl.program_id(0)
    qp = jax.lax.dot_general(
        q_ref[...].astype(BF16), wq_ref[...], (((1,), (1,)), ((), ())),
        preferred_element_type=F32) + bq_ref[...]
    b = b_ref[0, 0, :]
    sp = kp_ref.shape[0]
    row = i * R + jax.lax.broadcasted_iota(jnp.int32, (R, 1), 0)
    lo8, fits, oh_w = _window(b, lo_ref, hi_ref, i, sp)

    @pl.when(i == 0)
    def _init():
        den_ref[...] = jnp.zeros_like(den_ref)

    @pl.when(fits)
    def _fast():
        kx = jnp.dot(oh_w, kp_ref[pl.ds(lo8, W), :],
                     preferred_element_type=F32)
        ex = jnp.where(row < n, jnp.exp(qp * kx * scale), 0.0).astype(BF16)
        ex_ref[...] = ex
        den_ref[pl.ds(lo8, W), :] += jax.lax.dot_general(
            oh_w, ex, (((0,), (0,)), ((), ())), preferred_element_type=F32)

    @pl.when(jnp.logical_not(fits))
    def _slow():
        seg = jax.lax.broadcasted_iota(jnp.int32, (b.shape[0], sp), 1)
        oh = (b[:, None] == seg).astype(BF16)
        kx = jnp.dot(oh, kp_ref[...], preferred_element_type=F32)
        ex = jnp.where(row < n, jnp.exp(qp * kx * scale), 0.0).astype(BF16)
        ex_ref[...] = ex
        den_ref[...] += jax.lax.dot_general(
            oh, ex, (((0,), (0,)), ((), ())), preferred_element_type=F32)


def _pass2_body(lo_ref, hi_ref, ex_ref, b_ref, vp_ref, den_ref, wo_ref,
                bo_ref, out_ref):
    i = pl.program_id(0)
    b = b_ref[0, 0, :]
    sp = vp_ref.shape[0]
    lo8, fits, oh_w = _window(b, lo_ref, hi_ref, i, sp)

    def emit(wx):
        y = (ex_ref[...].astype(F32) * wx).astype(BF16)
        out_ref[...] = jax.lax.dot_general(
            y, wo_ref[...], (((1,), (1,)), ((), ())),
            preferred_element_type=F32) + bo_ref[...]

    @pl.when(fits)
    def _fast():
        den = den_ref[pl.ds(lo8, W), :]
        w = jnp.where(den > 0.0,
                      vp_ref[pl.ds(lo8, W), :].astype(F32) / den,
                      0.0).astype(BF16)
        emit(jnp.dot(oh_w, w, preferred_element_type=F32))

    @pl.when(jnp.logical_not(fits))
    def _slow():
        den = den_ref[...]
        w = jnp.where(den > 0.0, vp_ref[...].astype(F32) / den,
                      0.0).astype(BF16)
        seg = jax.lax.broadcasted_iota(jnp.int32, (b.shape[0], sp), 1)
        oh = (b[:, None] == seg).astype(BF16)
        emit(jnp.dot(oh, w, preferred_element_type=F32))


def kernel(q, k, v, batch, Wq, bq, Wk, bk, Wv, bv, Wo, bo):
    n, dm = q.shape
    s = k.shape[0]
    d = dm // H
    scale = 1.0 / math.sqrt(float(d))

    R = 2048
    nb = -(-n // R)
    npad = nb * R
    sp = -(-s // 128) * 128

    bi = batch.astype(jnp.int32)
    bz = jnp.pad(bi, (0, npad - n), mode="edge")
    b3 = bz.reshape(nb, 1, R)
    b2 = bz.reshape(nb, R)
    lo = b2[:, 0]
    hi = b2[:, R - 1]
    kz = jnp.pad(k, ((0, sp - s), (0, 0)))
    vz = jnp.pad(v, ((0, sp - s), (0, 0)))
    bq2, bk2, bv2, bo2 = (x.reshape(1, dm) for x in (bq, bk, bv, bo))

    full = lambda *shape: pl.BlockSpec(shape, lambda i: (0,) * len(shape))
    smem = pl.BlockSpec(memory_space=pltpu.SMEM)

    kp, vp = pl.pallas_call(
        _proj_body,
        grid=(1,),
        in_specs=[full(sp, dm), full(sp, dm), full(dm, dm), full(1, dm),
                  full(dm, dm), full(1, dm)],
        out_specs=[full(sp, dm), full(sp, dm)],
        out_shape=[jax.ShapeDtypeStruct((sp, dm), BF16),
                   jax.ShapeDtypeStruct((sp, dm), BF16)],
    )(kz, vz, Wk, bk2, Wv, bv2)

    ex, den = pl.pallas_call(
        functools.partial(_pass1_body, scale, R, n),
        grid=(nb,),
        in_specs=[
            smem, smem,
            pl.BlockSpec((R, dm), lambda i: (i, 0)),
            pl.BlockSpec((1, 1, R), lambda i: (i, 0, 0)),
            full(dm, dm), full(1, dm), full(sp, dm),
        ],
        out_specs=[pl.BlockSpec((R, dm), lambda i: (i, 0)), full(sp, dm)],
        out_shape=[jax.ShapeDtypeStruct((n, dm), BF16),
                   jax.ShapeDtypeStruct((sp, dm), F32)],
        compiler_params=pltpu.CompilerParams(
            dimension_semantics=("arbitrary",)),
    )(lo, hi, q, b3, Wq.astype(BF16), bq2, kp)

    out = pl.pallas_call(
        functools.partial(_pass2_body),
        grid=(nb,),
        in_specs=[
            smem, smem,
            pl.BlockSpec((R, dm), lambda i: (i, 0)),
            pl.BlockSpec((1, 1, R), lambda i: (i, 0, 0)),
            full(sp, dm), full(sp, dm), full(dm, dm), full(1, dm),
        ],
        out_specs=pl.BlockSpec((R, dm), lambda i: (i, 0)),
        out_shape=jax.ShapeDtypeStruct((n, dm), F32),
        compiler_params=pltpu.CompilerParams(
            dimension_semantics=("arbitrary",)),
    )(lo, hi, ex, b3, vp, den, Wo.astype(BF16), bo2)

    return out


# speedup vs baseline: 3.2398x; 1.0824x over previous
import functools
import math

import jax
import jax.numpy as jnp
from jax.experimental import pallas as pl
from jax.experimental.pallas import tpu as pltpu

H = 16
W = 128
F32 = jnp.float32
BF16 = jnp.bfloat16


def _proj_body(k_ref, v_ref, wk_ref, bk_ref, wv_ref, bv_ref, kp_ref, vp_ref):
    kp_ref[...] = (jax.lax.dot_general(
        k_ref[...], wk_ref[...], (((1,), (1,)), ((), ())),
        preferred_element_type=F32) + bk_ref[...]).astype(BF16)
    vp_ref[...] = (jax.lax.dot_general(
        v_ref[...], wv_ref[...], (((1,), (1,)), ((), ())),
        preferred_element_type=F32) + bv_ref[...]).astype(BF16)


def _window(b, lo_ref, hi_ref, i, sp):
    lo8 = pl.multiple_of(
        jnp.minimum((lo_ref[i] // 8) * 8, sp - W), 8)
    fits = hi_ref[i] - lo8 < W
    seg_w = lo8 + jax.lax.broadcasted_iota(jnp.int32, (b.shape[0], W), 1)
    oh_w = (b[:, None] == seg_w).astype(BF16)
    return lo8, fits, oh_w


def _pass1_body(scale, R, n, lo_ref, hi_ref, q_ref, b_ref, wq_ref, bq_ref,
                kp_ref, ex_ref, den_ref):
    i = pl.program_id(0)
    qp = jax.lax.dot_general(
        q_ref[...].astype(BF16), wq_ref[...], (((1,), (1,)), ((), ())),
        preferred_element_type=F32) + bq_ref[...]
    b = b_ref[0, 0, :]
    sp = kp_ref.shape[0]
    row = i * R + jax.lax.broadcasted_iota(jnp.int32, (R, 1), 0)
    lo8, fits, oh_w = _window(b, lo_ref, hi_ref, i, sp)

    @pl.when(i == 0)
    def _init():
        den_ref[...] = jnp.zeros_like(den_ref)

    @pl.when(fits)
    def _fast():
        kx = jnp.dot(oh_w, kp_ref[pl.ds(lo8, W), :],
                     preferred_element_type=F32)
        ex = jnp.where(row < n, jnp.exp(qp * kx * scale), 0.0).astype(BF16)
        ex_ref[...] = ex
        den_ref[pl.ds(lo8, W), :] += jax.lax.dot_general(
            oh_w, ex, (((0,), (0,)), ((), ())), preferred_element_type=F32)

    @pl.when(jnp.logical_not(fits))
    def _slow():
        seg = jax.lax.broadcasted_iota(jnp.int32, (b.shape[0], sp), 1)
        oh = (b[:, None] == seg).astype(BF16)
        kx = jnp.dot(oh, kp_ref[...], preferred_element_type=F32)
        ex = jnp.where(row < n, jnp.exp(qp * kx * scale), 0.0).astype(BF16)
        ex_ref[...] = ex
        den_ref[...] += jax.lax.dot_general(
            oh, ex, (((0,), (0,)), ((), ())), preferred_element_type=F32)


def _pass2_body(lo_ref, hi_ref, ex_ref, b_ref, vp_ref, den_ref, wo_ref,
                bo_ref, out_ref):
    i = pl.program_id(0)
    b = b_ref[0, 0, :]
    sp = vp_ref.shape[0]
    lo8, fits, oh_w = _window(b, lo_ref, hi_ref, i, sp)

    def emit(wx):
        y = (ex_ref[...].astype(F32) * wx).astype(BF16)
        out_ref[...] = jax.lax.dot_general(
            y, wo_ref[...], (((1,), (1,)), ((), ())),
            preferred_element_type=F32) + bo_ref[...]

    @pl.when(fits)
    def _fast():
        den = den_ref[pl.ds(lo8, W), :]
        w = jnp.where(den > 0.0,
                      vp_ref[pl.ds(lo8, W), :].astype(F32) / den,
                      0.0).astype(BF16)
        emit(jnp.dot(oh_w, w, preferred_element_type=F32))

    @pl.when(jnp.logical_not(fits))
    def _slow():
        den = den_ref[...]
        w = jnp.where(den > 0.0, vp_ref[...].astype(F32) / den,
                      0.0).astype(BF16)
        seg = jax.lax.broadcasted_iota(jnp.int32, (b.shape[0], sp), 1)
        oh = (b[:, None] == seg).astype(BF16)
        emit(jnp.dot(oh, w, preferred_element_type=F32))


def kernel(q, k, v, batch, Wq, bq, Wk, bk, Wv, bv, Wo, bo):
    n, dm = q.shape
    s = k.shape[0]
    d = dm // H
    scale = 1.0 / math.sqrt(float(d))

    R = 4096
    nb = -(-n // R)
    npad = nb * R
    sp = -(-s // 128) * 128

    bi = batch.astype(jnp.int32)
    bz = jnp.pad(bi, (0, npad - n), mode="edge")
    b3 = bz.reshape(nb, 1, R)
    b2 = bz.reshape(nb, R)
    lo = b2[:, 0]
    hi = b2[:, R - 1]
    kz = jnp.pad(k, ((0, sp - s), (0, 0)))
    vz = jnp.pad(v, ((0, sp - s), (0, 0)))
    bq2, bk2, bv2, bo2 = (x.reshape(1, dm) for x in (bq, bk, bv, bo))

    full = lambda *shape: pl.BlockSpec(shape, lambda i: (0,) * len(shape))
    smem = pl.BlockSpec(memory_space=pltpu.SMEM)

    kp, vp = pl.pallas_call(
        _proj_body,
        grid=(1,),
        in_specs=[full(sp, dm), full(sp, dm), full(dm, dm), full(1, dm),
                  full(dm, dm), full(1, dm)],
        out_specs=[full(sp, dm), full(sp, dm)],
        out_shape=[jax.ShapeDtypeStruct((sp, dm), BF16),
                   jax.ShapeDtypeStruct((sp, dm), BF16)],
    )(kz, vz, Wk, bk2, Wv, bv2)

    ex, den = pl.pallas_call(
        functools.partial(_pass1_body, scale, R, n),
        grid=(nb,),
        in_specs=[
            smem, smem,
            pl.BlockSpec((R, dm), lambda i: (i, 0)),
            pl.BlockSpec((1, 1, R), lambda i: (i, 0, 0)),
            full(dm, dm), full(1, dm), full(sp, dm),
        ],
        out_specs=[pl.BlockSpec((R, dm), lambda i: (i, 0)), full(sp, dm)],
        out_shape=[jax.ShapeDtypeStruct((n, dm), BF16),
                   jax.ShapeDtypeStruct((sp, dm), F32)],
        compiler_params=pltpu.CompilerParams(
            dimension_semantics=("arbitrary",)),
    )(lo, hi, q, b3, Wq.astype(BF16), bq2, kp)

    out = pl.pallas_call(
        functools.partial(_pass2_body),
        grid=(nb,),
        in_specs=[
            smem, smem,
            pl.BlockSpec((R, dm), lambda i: (i, 0)),
            pl.BlockSpec((1, 1, R), lambda i: (i, 0, 0)),
            full(sp, dm), full(sp, dm), full(dm, dm), full(1, dm),
        ],
        out_specs=pl.BlockSpec((R, dm), lambda i: (i, 0)),
        out_shape=jax.ShapeDtypeStruct((n, dm), F32),
        compiler_params=pltpu.CompilerParams(
            dimension_semantics=("arbitrary",)),
    )(lo, hi, ex, b3, vp, den, Wo.astype(BF16), bo2)

    return out


# speedup vs baseline: 3.3363x; 1.0298x over previous
import functools
import math

import jax
import jax.numpy as jnp
from jax.experimental import pallas as pl
from jax.experimental.pallas import tpu as pltpu

H = 16
W = 128
F32 = jnp.float32
BF16 = jnp.bfloat16


def _proj_body(k_ref, v_ref, wk_ref, bk_ref, wv_ref, bv_ref, kp_ref, vp_ref):
    kp_ref[...] = (jax.lax.dot_general(
        k_ref[...], wk_ref[...], (((1,), (1,)), ((), ())),
        preferred_element_type=F32) + bk_ref[...]).astype(BF16)
    vp_ref[...] = (jax.lax.dot_general(
        v_ref[...], wv_ref[...], (((1,), (1,)), ((), ())),
        preferred_element_type=F32) + bv_ref[...]).astype(BF16)


def _window(b, lo_ref, hi_ref, i, sp):
    lo8 = pl.multiple_of(
        jnp.minimum((lo_ref[i] // 8) * 8, sp - W), 8)
    fits = hi_ref[i] - lo8 < W
    seg_w = lo8 + jax.lax.broadcasted_iota(jnp.int32, (b.shape[0], W), 1)
    oh_w = (b[:, None] == seg_w).astype(BF16)
    return lo8, fits, oh_w


def _pass1_body(scale, R, n, lo_ref, hi_ref, q_ref, b_ref, wq_ref, bq_ref,
                kp_ref, ex_ref, den_ref):
    i = pl.program_id(0)
    qp = jax.lax.dot_general(
        q_ref[...].astype(BF16), wq_ref[...], (((1,), (1,)), ((), ())),
        preferred_element_type=F32) + bq_ref[...]
    b = b_ref[0, 0, :]
    sp = kp_ref.shape[0]
    row = i * R + jax.lax.broadcasted_iota(jnp.int32, (R, 1), 0)
    lo8, fits, oh_w = _window(b, lo_ref, hi_ref, i, sp)

    @pl.when(i == 0)
    def _init():
        den_ref[...] = jnp.zeros_like(den_ref)

    @pl.when(fits)
    def _fast():
        kx = jnp.dot(oh_w, kp_ref[pl.ds(lo8, W), :],
                     preferred_element_type=F32)
        ex = jnp.where(row < n, jnp.exp(qp * kx * scale), 0.0).astype(BF16)
        ex_ref[...] = ex
        den_ref[pl.ds(lo8, W), :] += jax.lax.dot_general(
            oh_w, ex, (((0,), (0,)), ((), ())), preferred_element_type=F32)

    @pl.when(jnp.logical_not(fits))
    def _slow():
        seg = jax.lax.broadcasted_iota(jnp.int32, (b.shape[0], sp), 1)
        oh = (b[:, None] == seg).astype(BF16)
        kx = jnp.dot(oh, kp_ref[...], preferred_element_type=F32)
        ex = jnp.where(row < n, jnp.exp(qp * kx * scale), 0.0).astype(BF16)
        ex_ref[...] = ex
        den_ref[...] += jax.lax.dot_general(
            oh, ex, (((0,), (0,)), ((), ())), preferred_element_type=F32)


def _pass2_body(lo_ref, hi_ref, ex_ref, b_ref, vp_ref, den_ref, wo_ref,
                bo_ref, out_ref):
    i = pl.program_id(0)
    b = b_ref[0, 0, :]
    sp = vp_ref.shape[0]
    lo8, fits, oh_w = _window(b, lo_ref, hi_ref, i, sp)

    def emit(wx):
        y = (ex_ref[...].astype(F32) * wx).astype(BF16)
        out_ref[...] = jax.lax.dot_general(
            y, wo_ref[...], (((1,), (1,)), ((), ())),
            preferred_element_type=F32) + bo_ref[...]

    @pl.when(fits)
    def _fast():
        den = den_ref[pl.ds(lo8, W), :]
        w = jnp.where(den > 0.0,
                      vp_ref[pl.ds(lo8, W), :].astype(F32) / den,
                      0.0).astype(BF16)
        emit(jnp.dot(oh_w, w, preferred_element_type=F32))

    @pl.when(jnp.logical_not(fits))
    def _slow():
        den = den_ref[...]
        w = jnp.where(den > 0.0, vp_ref[...].astype(F32) / den,
                      0.0).astype(BF16)
        seg = jax.lax.broadcasted_iota(jnp.int32, (b.shape[0], sp), 1)
        oh = (b[:, None] == seg).astype(BF16)
        emit(jnp.dot(oh, w, preferred_element_type=F32))


def kernel(q, k, v, batch, Wq, bq, Wk, bk, Wv, bv, Wo, bo):
    n, dm = q.shape
    s = k.shape[0]
    d = dm // H
    scale = 1.0 / math.sqrt(float(d))

    R = 5120
    nb = -(-n // R)
    npad = nb * R
    sp = -(-s // 128) * 128

    bi = batch.astype(jnp.int32)
    bz = jnp.pad(bi, (0, npad - n), mode="edge")
    b3 = bz.reshape(nb, 1, R)
    b2 = bz.reshape(nb, R)
    lo = b2[:, 0]
    hi = b2[:, R - 1]
    kz = jnp.pad(k, ((0, sp - s), (0, 0)))
    vz = jnp.pad(v, ((0, sp - s), (0, 0)))
    bq2, bk2, bv2, bo2 = (x.reshape(1, dm) for x in (bq, bk, bv, bo))

    full = lambda *shape: pl.BlockSpec(shape, lambda i: (0,) * len(shape))
    smem = pl.BlockSpec(memory_space=pltpu.SMEM)

    kp, vp = pl.pallas_call(
        _proj_body,
        grid=(1,),
        in_specs=[full(sp, dm), full(sp, dm), full(dm, dm), full(1, dm),
                  full(dm, dm), full(1, dm)],
        out_specs=[full(sp, dm), full(sp, dm)],
        out_shape=[jax.ShapeDtypeStruct((sp, dm), BF16),
                   jax.ShapeDtypeStruct((sp, dm), BF16)],
    )(kz, vz, Wk, bk2, Wv, bv2)

    ex, den = pl.pallas_call(
        functools.partial(_pass1_body, scale, R, n),
        grid=(nb,),
        in_specs=[
            smem, smem,
            pl.BlockSpec((R, dm), lambda i: (i, 0)),
            pl.BlockSpec((1, 1, R), lambda i: (i, 0, 0)),
            full(dm, dm), full(1, dm), full(sp, dm),
        ],
        out_specs=[pl.BlockSpec((R, dm), lambda i: (i, 0)), full(sp, dm)],
        out_shape=[jax.ShapeDtypeStruct((n, dm), BF16),
                   jax.ShapeDtypeStruct((sp, dm), F32)],
        compiler_params=pltpu.CompilerParams(
            dimension_semantics=("arbitrary",),
            vmem_limit_bytes=66060288),
    )(lo, hi, q, b3, Wq.astype(BF16), bq2, kp)

    out = pl.pallas_call(
        functools.partial(_pass2_body),
        grid=(nb,),
        in_specs=[
            smem, smem,
            pl.BlockSpec((R, dm), lambda i: (i, 0)),
            pl.BlockSpec((1, 1, R), lambda i: (i, 0, 0)),
            full(sp, dm), full(sp, dm), full(dm, dm), full(1, dm),
        ],
        out_specs=pl.BlockSpec((R, dm), lambda i: (i, 0)),
        out_shape=jax.ShapeDtypeStruct((n, dm), F32),
        compiler_params=pltpu.CompilerParams(
            dimension_semantics=("arbitrary",),
            vmem_limit_bytes=66060288),
    )(lo, hi, ex, b3, vp, den, Wo.astype(BF16), bo2)

    return out
